# baseline (speedup 1.0000x reference)
import jax
import jax.numpy as jnp
from jax.experimental import pallas as pl
from jax.experimental.pallas import tpu as pltpu

_DIM = 770
_KQ = 65536
_B = 4096
_EMB = 768


def _body(p_ref, emb_ref, extra_ref, q_ref, o_ref):
    del p_ref, q_ref
    o_ref[0:_EMB, :] = emb_ref[...].T
    o_ref[_EMB:_DIM, :] = extra_ref[...]


def kernel(embedding_batch, CLabel, NumofLabel, queue, queue_ptr):
    n = embedding_batch.shape[0]
    idx = jnp.arange(n, dtype=jnp.float32) + (
        jnp.asarray(NumofLabel, dtype=jnp.float32) - jnp.float32(n)
    )
    extra = jnp.stack([CLabel.astype(jnp.float32), idx])

    ptr = queue_ptr[0]
    p = jnp.where(ptr + _B >= _KQ - 1, jnp.int32(0), ptr).astype(jnp.int32)
    pb = (p // _B).reshape(1)

    grid_spec = pltpu.PrefetchScalarGridSpec(
        num_scalar_prefetch=1,
        grid=(1,),
        in_specs=[
            pl.BlockSpec((n, _EMB), lambda i, pb: (0, 0)),
            pl.BlockSpec((2, _B), lambda i, pb: (0, 0)),
            pl.BlockSpec(memory_space=pl.ANY),
        ],
        out_specs=pl.BlockSpec((_DIM, _B), lambda i, pb: (0, pb[0])),
    )

    out = pl.pallas_call(
        _body,
        grid_spec=grid_spec,
        out_shape=jax.ShapeDtypeStruct((_DIM, _KQ), jnp.float32),
        input_output_aliases={3: 0},
    )(pb, embedding_batch, extra, queue)

    new_ptr = p + jnp.int32(_B)
    return (out, new_ptr)


# speedup vs baseline: 1.0539x; 1.0539x over previous
import jax
import jax.numpy as jnp
from jax.experimental import pallas as pl
from jax.experimental.pallas import tpu as pltpu

_DIM = 770
_KQ = 65536
_B = 4096
_EMB = 768
_C = 2048
_NC = _KQ // _C


def _body(pb_ref, emb_ref, extra_ref, q_ref, o_ref):
    i = pl.program_id(0)
    c0 = pb_ref[0]
    is_upd = (i >= c0) & (i < c0 + 2)

    @pl.when(is_upd)
    def _():
        o_ref[0:_EMB, :] = emb_ref[...].T
        off = pl.multiple_of((i - c0) * _C, _C)
        o_ref[_EMB:_DIM, :] = extra_ref[:, pl.ds(off, _C)]

    @pl.when(jnp.logical_not(is_upd))
    def _():
        o_ref[...] = q_ref[...]


def kernel(embedding_batch, CLabel, NumofLabel, queue, queue_ptr):
    n = embedding_batch.shape[0]
    idx = jnp.arange(n, dtype=jnp.float32) + (
        jnp.asarray(NumofLabel, dtype=jnp.float32) - jnp.float32(n)
    )
    extra = jnp.stack([CLabel.astype(jnp.float32), idx])

    ptr = queue_ptr[0]
    p = jnp.where(ptr + _B >= _KQ - 1, jnp.int32(0), ptr).astype(jnp.int32)
    pb = (p // _C).reshape(1)

    def emb_map(j, pb):
        return (jnp.clip(j - pb[0], 0, 1), 0)

    def q_map(j, pb):
        c0 = pb[0]
        is_upd = (j >= c0) & (j < c0 + 2)
        return (0, jnp.where(is_upd, c0 + 2, j))

    grid_spec = pltpu.PrefetchScalarGridSpec(
        num_scalar_prefetch=1,
        grid=(_NC,),
        in_specs=[
            pl.BlockSpec((_C, _EMB), emb_map),
            pl.BlockSpec((2, _B), lambda j, pb: (0, 0)),
            pl.BlockSpec((_DIM, _C), q_map),
        ],
        out_specs=pl.BlockSpec((_DIM, _C), lambda j, pb: (0, j)),
    )

    out = pl.pallas_call(
        _body,
        grid_spec=grid_spec,
        out_shape=jax.ShapeDtypeStruct((_DIM, _KQ), jnp.float32),
        compiler_params=pltpu.CompilerParams(
            dimension_semantics=("arbitrary",),
        ),
    )(pb, embedding_batch, extra, queue)

    new_ptr = p + jnp.int32(_B)
    return (out, new_ptr)
